# Initial kernel scaffold; baseline (speedup 1.0000x reference)
import jax
import jax.numpy as jnp
from jax.experimental import pallas as pl


def kernel(h, edge_index, rel_ids, W):
    raise NotImplementedError("write your pallas kernel here")


# speedup vs baseline: 1.3258x; 1.3258x over previous
import jax
import jax.numpy as jnp
from jax import lax
from jax.experimental import pallas as pl
from jax.experimental.pallas import tpu as pltpu
from jax.experimental.pallas import tpu_sc as plsc

_E = 320000
_D = 128
_NC = 2
_NS = 16
_NW = _NC * _NS
_EPW = _E // _NW
_C = 80
_NCHUNK = _EPW // _C

_GATHER_DNUMS = lax.GatherDimensionNumbers(
    offset_dims=(), collapsed_slice_dims=(0,), start_index_map=(0,))


def _lane_shuffle(v, idx):
    return lax.gather(v, idx[:, None], _GATHER_DNUMS, slice_sizes=(1,),
                      mode=lax.GatherScatterMode.PROMISE_IN_BOUNDS)


def _body(h_hbm, w_hbm, src_hbm, dst_hbm, rel_hbm, out_hbm,
          srcv, dstv, relv, rows_u, rows_v, rows_w, outv, sem):
    wid = lax.axis_index("s") * _NC + lax.axis_index("c")
    lane = lax.iota(jnp.int32, 16)
    shuf = [(lane + sh) % 16 for sh in (8, 4, 2, 1)]

    def chunk_body(g, carry):
        base = wid * _EPW + g * _C
        pltpu.sync_copy(src_hbm.at[pl.ds(base, _C)], srcv)
        pltpu.sync_copy(dst_hbm.at[pl.ds(base, _C)], dstv)
        pltpu.sync_copy(rel_hbm.at[pl.ds(base, _C)], relv)
        pltpu.async_copy(h_hbm.at[srcv], rows_u, sem).wait()
        pltpu.async_copy(h_hbm.at[dstv], rows_v, sem).wait()
        pltpu.async_copy(w_hbm.at[relv], rows_w, sem).wait()

        def group_body(t, c2):
            e0 = t * 16
            scores = jnp.zeros((16,), jnp.float32)
            for i in range(16):
                e = e0 + i
                acc = jnp.zeros((16,), jnp.float32)
                for j in range(_D // 16):
                    u = rows_u[e, pl.ds(j * 16, 16)]
                    v = rows_v[e, pl.ds(j * 16, 16)]
                    w = rows_w[e, pl.ds(j * 16, 16)]
                    acc = acc + u * v * w
                for s in shuf:
                    acc = acc + _lane_shuffle(acc, s)
                scores = jnp.where(lane == i, acc, scores)
            outv[pl.ds(e0, 16)] = scores
            return c2

        lax.fori_loop(0, _C // 16, group_body, 0)
        pltpu.sync_copy(outv, out_hbm.at[pl.ds(base, _C)])
        return carry

    lax.fori_loop(0, _NCHUNK, chunk_body, 0)


def kernel(h, edge_index, rel_ids, W):
    src = edge_index[0].astype(jnp.int32)
    dst = edge_index[1].astype(jnp.int32)
    rel = rel_ids.astype(jnp.int32)
    mesh = plsc.VectorSubcoreMesh(core_axis_name="c", subcore_axis_name="s")
    k = pl.kernel(
        _body,
        mesh=mesh,
        out_type=jax.ShapeDtypeStruct((_E,), jnp.float32),
        scratch_types=[
            pltpu.VMEM((_C,), jnp.int32),
            pltpu.VMEM((_C,), jnp.int32),
            pltpu.VMEM((_C,), jnp.int32),
            pltpu.VMEM((_C, _D), jnp.float32),
            pltpu.VMEM((_C, _D), jnp.float32),
            pltpu.VMEM((_C, _D), jnp.float32),
            pltpu.VMEM((_C,), jnp.float32),
            pltpu.SemaphoreType.DMA,
        ],
    )
    return k(h, W, src, dst, rel)


# speedup vs baseline: 7.1447x; 5.3889x over previous
import jax
import jax.numpy as jnp
from jax import lax
from jax.experimental import pallas as pl
from jax.experimental.pallas import tpu as pltpu
from jax.experimental.pallas import tpu_sc as plsc

_N = 10000
_E = 320000
_D = 128
_R = 8
_NC = 2
_NS = 16
_NW = _NC * _NS
_EPW = _E // _NW
_C = 80
_NCHUNK = _EPW // _C
_NBUF = 3

_GATHER_DNUMS = lax.GatherDimensionNumbers(
    offset_dims=(), collapsed_slice_dims=(0,), start_index_map=(0,))


def _lane_shuffle(v, idx):
    return lax.gather(v, idx[:, None], _GATHER_DNUMS, slice_sizes=(1,),
                      mode=lax.GatherScatterMode.PROMISE_IN_BOUNDS)


def _scale_body(h_ref, w_ref, g_ref):
    r = pl.program_id(0)
    g_ref[...] = (h_ref[...] * w_ref[pl.ds(r, 1), :])[None]


def _prescale(h, W):
    g3 = pl.pallas_call(
        _scale_body,
        grid=(_R,),
        in_specs=[
            pl.BlockSpec((_N, _D), lambda r: (0, 0)),
            pl.BlockSpec((_R, _D), lambda r: (0, 0)),
        ],
        out_specs=pl.BlockSpec((1, _N, _D), lambda r: (r, 0, 0)),
        out_shape=jax.ShapeDtypeStruct((_R, _N, _D), jnp.float32),
    )(h, W)
    return g3.reshape(_R * _N, _D)


def _sc_body(g_hbm, h_hbm, idxu_hbm, idxv_hbm, out_hbm,
             idxu_v, idxv_v, outv,
             ru0, ru1, ru2, rv0, rv1, rv2, sem0, sem1, sem2):
    wid = lax.axis_index("s") * _NC + lax.axis_index("c")
    lane = lax.iota(jnp.int32, 16)
    shuf = [(lane + sh) % 16 for sh in (8, 4, 2, 1)]
    rubufs = (ru0, ru1, ru2)
    rvbufs = (rv0, rv1, rv2)
    sems = (sem0, sem1, sem2)

    base = wid * _EPW
    pltpu.sync_copy(idxu_hbm.at[pl.ds(base, _EPW)], idxu_v)
    pltpu.sync_copy(idxv_hbm.at[pl.ds(base, _EPW)], idxv_v)

    def fire(g, b):
        off = g * _C
        pltpu.async_copy(g_hbm.at[idxu_v.at[pl.ds(off, _C)]], rubufs[b],
                         sems[b])
        pltpu.async_copy(h_hbm.at[idxv_v.at[pl.ds(off, _C)]], rvbufs[b],
                         sems[b])

    def drain(g, b):
        pltpu.make_async_copy(g_hbm.at[idxu_v.at[pl.ds(0, _C)]], rubufs[b],
                              sems[b]).wait()
        pltpu.make_async_copy(h_hbm.at[idxv_v.at[pl.ds(0, _C)]], rvbufs[b],
                              sems[b]).wait()

    def compute(g, b):
        ru, rv = rubufs[b], rvbufs[b]

        def group_body(t, c2):
            e0 = t * 16
            scores = jnp.zeros((16,), jnp.float32)
            for i in range(16):
                e = e0 + i
                acc = jnp.zeros((16,), jnp.float32)
                for j in range(_D // 16):
                    u = ru[e, pl.ds(j * 16, 16)]
                    v = rv[e, pl.ds(j * 16, 16)]
                    acc = acc + u * v
                for s in shuf:
                    acc = acc + _lane_shuffle(acc, s)
                scores = jnp.where(lane == i, acc, scores)
            outv[pl.ds(g * _C + e0, 16)] = scores
            return c2

        lax.fori_loop(0, _C // 16, group_body, 0)

    for b in range(_NBUF - 1):
        fire(b, b)

    nsteady = _NCHUNK - (_NBUF - 1)
    nloops = nsteady // _NBUF

    def ring_body(k, carry):
        for i in range(_NBUF):
            g = k * _NBUF + i
            drain(g, i)
            fire(g + _NBUF - 1, (i + _NBUF - 1) % _NBUF)
            compute(g, i)
        return carry

    lax.fori_loop(0, nloops, ring_body, 0)

    for g in range(nloops * _NBUF, _NCHUNK):
        b = g % _NBUF
        drain(g, b)
        compute(g, b)

    pltpu.sync_copy(outv, out_hbm.at[pl.ds(base, _EPW)])


def kernel(h, edge_index, rel_ids, W):
    src = edge_index[0].astype(jnp.int32)
    dst = edge_index[1].astype(jnp.int32)
    rel = rel_ids.astype(jnp.int32)
    idx_u = rel * _N + src
    gtab = _prescale(h, W)
    mesh = plsc.VectorSubcoreMesh(core_axis_name="c", subcore_axis_name="s")
    k = pl.kernel(
        _sc_body,
        mesh=mesh,
        out_type=jax.ShapeDtypeStruct((_E,), jnp.float32),
        scratch_types=[
            pltpu.VMEM((_EPW,), jnp.int32),
            pltpu.VMEM((_EPW,), jnp.int32),
            pltpu.VMEM((_EPW,), jnp.float32),
            pltpu.VMEM((_C, _D), jnp.float32),
            pltpu.VMEM((_C, _D), jnp.float32),
            pltpu.VMEM((_C, _D), jnp.float32),
            pltpu.VMEM((_C, _D), jnp.float32),
            pltpu.VMEM((_C, _D), jnp.float32),
            pltpu.VMEM((_C, _D), jnp.float32),
            pltpu.SemaphoreType.DMA,
            pltpu.SemaphoreType.DMA,
            pltpu.SemaphoreType.DMA,
        ],
    )
    return k(gtab, h, idx_u, dst)
